# baseline (device time: 19148 ns/iter reference)
import jax
import jax.numpy as jnp
from jax import lax
from jax.experimental import pallas as pl
from jax.experimental.pallas import tpu as pltpu

N_DEV = 4
B, SQ, SKV, D = 2, 256, 256, 512
HL, DH = 4, 64
HD = HL * DH
BLK = 64
NP = HL // 2
PW = 2 * DH


def kernel(x, Wq, K_ext, V_ext, Wo):
    xf = x.reshape(B * SQ, D)

    def body(x_ref, wq_ref, k_ref, v_ref, wo_ref, out_ref,
             wq_scr, kscr, vscr, ctx_mine, ctx_recv,
             wqsem, ksems, vsems, ssems, rsems):
        my_pos = lax.axis_index("i")

        wq_dma = pltpu.make_async_copy(
            wq_ref.at[:, pl.ds(my_pos * HD, HD)], wq_scr, wqsem)
        wq_dma.start()

        def kv_dma(ref, scr, sems, b, h):
            return pltpu.make_async_copy(
                ref.at[b, :, h, :], scr.at[b, h], sems.at[b, h])

        for b in range(B):
            for h in range(HL):
                kv_dma(k_ref, kscr, ksems, b, h).start()
                kv_dma(v_ref, vscr, vsems, b, h).start()

        barrier_sem = pltpu.get_barrier_semaphore()
        for d in (1, 2, 3):
            pl.semaphore_signal(
                barrier_sem, inc=1,
                device_id=(lax.rem(my_pos + d, N_DEV),),
                device_id_type=pl.DeviceIdType.MESH,
            )

        wq_dma.wait()
        q_all = jnp.dot(x_ref[...], wq_scr[...],
                        preferred_element_type=jnp.float32)

        def make_desc(k, b, p, dev_offset):
            return pltpu.make_async_remote_copy(
                src_ref=ctx_mine.at[b, :, pl.ds(p * PW, PW)],
                dst_ref=ctx_recv.at[k, b, :, pl.ds(p * PW, PW)],
                send_sem=ssems.at[k, b, p],
                recv_sem=rsems.at[k, b, p],
                device_id=(lax.rem(my_pos + dev_offset, N_DEV),),
                device_id_type=pl.DeviceIdType.MESH,
            )

        def softmax_ctx(q, kk, vv):
            s = lax.dot_general(q, kk, (((1,), (1,)), ((), ())),
                                preferred_element_type=jnp.float32)
            w = jnp.exp(s * 0.125)
            w = w / jnp.sum(w, axis=1, keepdims=True)
            return jnp.dot(w, vv, preferred_element_type=jnp.float32)

        first = True
        for b in range(B):
            for p in range(NP):
                pair_parts = []
                for h in (2 * p, 2 * p + 1):
                    kv_dma(k_ref, kscr, ksems, b, h).wait()
                    kv_dma(v_ref, vscr, vsems, b, h).wait()
                    qh = q_all[b * SQ:(b + 1) * SQ, h * DH:(h + 1) * DH]
                    kh = kscr[b, h]
                    vh = vscr[b, h]
                    qa = jnp.concatenate([qh[:BLK], qh[3 * BLK:]], axis=0)
                    ka = jnp.concatenate([kh[:BLK], kh[3 * BLK:]], axis=0)
                    va = jnp.concatenate([vh[:BLK], vh[3 * BLK:]], axis=0)
                    ctx_a = softmax_ctx(qa, ka, va)
                    ctx_b = softmax_ctx(qh[BLK:3 * BLK],
                                        kh[:3 * BLK], vh[:3 * BLK])
                    pair_parts.append(jnp.concatenate(
                        [ctx_a[:BLK], ctx_b, ctx_a[BLK:]], axis=0))
                ctx_mine[b, :, pl.ds(p * PW, PW)] = jnp.concatenate(
                    pair_parts, axis=1).astype(jnp.bfloat16)
                if first:
                    pl.semaphore_wait(barrier_sem, 3)
                    first = False
                for d in (1, 2, 3):
                    make_desc(3 - d, b, p, d).start()

        wo_my = wo_ref[pl.ds(my_pos * HD, HD), :]
        for b in range(B):
            out_ref[b] = jnp.dot(ctx_mine[b].astype(jnp.float32), wo_my,
                                 preferred_element_type=jnp.float32)

        for k in (0, 2, 1):
            origin = lax.rem(my_pos + k + 1, N_DEV)
            wo_k = wo_ref[pl.ds(origin * HD, HD), :]
            for b in range(B):
                for p in range(NP):
                    make_desc(k, b, p, k + 1).wait_recv()
                out_ref[b] = out_ref[b] + jnp.dot(
                    ctx_recv[k, b].astype(jnp.float32), wo_k,
                    preferred_element_type=jnp.float32)

        for k in range(N_DEV - 1):
            for b in range(B):
                for p in range(NP):
                    make_desc(k, b, p, 3 - k).wait_send()

    return pl.pallas_call(
        body,
        out_shape=jax.ShapeDtypeStruct((B, SQ, D), jnp.float32),
        in_specs=[
            pl.BlockSpec(memory_space=pltpu.VMEM),
            pl.BlockSpec(memory_space=pl.ANY),
            pl.BlockSpec(memory_space=pl.ANY),
            pl.BlockSpec(memory_space=pl.ANY),
            pl.BlockSpec(memory_space=pltpu.VMEM),
        ],
        out_specs=pl.BlockSpec(memory_space=pltpu.VMEM),
        scratch_shapes=[
            pltpu.VMEM((D, HD), jnp.float32),
            pltpu.VMEM((B, HL, SKV, DH), jnp.float32),
            pltpu.VMEM((B, HL, SKV, DH), jnp.float32),
            pltpu.VMEM((B, SQ, HD), jnp.bfloat16),
            pltpu.VMEM((N_DEV - 1, B, SQ, HD), jnp.bfloat16),
            pltpu.SemaphoreType.DMA,
            pltpu.SemaphoreType.DMA((B, HL)),
            pltpu.SemaphoreType.DMA((B, HL)),
            pltpu.SemaphoreType.DMA((N_DEV - 1, B, NP)),
            pltpu.SemaphoreType.DMA((N_DEV - 1, B, NP)),
        ],
        compiler_params=pltpu.CompilerParams(collective_id=0),
    )(xf, Wq, K_ext, V_ext, Wo)


# device time: 14820 ns/iter; 1.2920x vs baseline; 1.2920x over previous
import jax
import jax.numpy as jnp
from jax import lax
from jax.experimental import pallas as pl
from jax.experimental.pallas import tpu as pltpu

N_DEV = 4
B, SQ, SKV, D = 2, 256, 256, 512
HL, DH = 4, 64
HD = HL * DH
BLK = 64
NP = HL // 2
PW = 2 * DH


def kernel(x, Wq, K_ext, V_ext, Wo):
    my = lax.axis_index("i")
    Wq_loc = lax.dynamic_slice_in_dim(Wq, my * HD, HD, axis=1)
    xf = x.reshape(B * SQ, D)
    Kh = K_ext.transpose(0, 2, 1, 3)
    Vh = V_ext.transpose(0, 2, 1, 3)

    def body(x_ref, wq_ref, k_ref, v_ref, wo_ref, out_ref,
             ctx_mine, ctx_recv, ssems, rsems):
        my_pos = lax.axis_index("i")

        barrier_sem = pltpu.get_barrier_semaphore()
        for d in (1, 2, 3):
            pl.semaphore_signal(
                barrier_sem, inc=1,
                device_id=(lax.rem(my_pos + d, N_DEV),),
                device_id_type=pl.DeviceIdType.MESH,
            )

        q_all = jnp.dot(x_ref[...], wq_ref[...],
                        preferred_element_type=jnp.float32)

        def make_desc(k, b, p, dev_offset):
            return pltpu.make_async_remote_copy(
                src_ref=ctx_mine.at[b, :, pl.ds(p * PW, PW)],
                dst_ref=ctx_recv.at[k, b, :, pl.ds(p * PW, PW)],
                send_sem=ssems.at[k, b, p],
                recv_sem=rsems.at[k, b, p],
                device_id=(lax.rem(my_pos + dev_offset, N_DEV),),
                device_id_type=pl.DeviceIdType.MESH,
            )

        def softmax_ctx(q, kk, vv):
            s = lax.dot_general(q, kk, (((1,), (1,)), ((), ())),
                                preferred_element_type=jnp.float32)
            w = jnp.exp(s * 0.125)
            w = w / jnp.sum(w, axis=1, keepdims=True)
            return jnp.dot(w, vv, preferred_element_type=jnp.float32)

        first = True
        for b in range(B):
            for p in range(NP):
                pair_parts = []
                for h in (2 * p, 2 * p + 1):
                    qh = q_all[b * SQ:(b + 1) * SQ, h * DH:(h + 1) * DH]
                    kh = k_ref[b, h]
                    vh = v_ref[b, h]
                    qa = jnp.concatenate([qh[:BLK], qh[3 * BLK:]], axis=0)
                    ka = jnp.concatenate([kh[:BLK], kh[3 * BLK:]], axis=0)
                    va = jnp.concatenate([vh[:BLK], vh[3 * BLK:]], axis=0)
                    ctx_a = softmax_ctx(qa, ka, va)
                    ctx_b = softmax_ctx(qh[BLK:3 * BLK],
                                        kh[:3 * BLK], vh[:3 * BLK])
                    pair_parts.append(jnp.concatenate(
                        [ctx_a[:BLK], ctx_b, ctx_a[BLK:]], axis=0))
                ctx_mine[b, :, pl.ds(p * PW, PW)] = jnp.concatenate(
                    pair_parts, axis=1).astype(jnp.bfloat16)
                if first:
                    pl.semaphore_wait(barrier_sem, 3)
                    first = False
                for d in (1, 2, 3):
                    make_desc(3 - d, b, p, d).start()

        wo_my = wo_ref[pl.ds(my_pos * HD, HD), :]
        for b in range(B):
            out_ref[b] = jnp.dot(ctx_mine[b].astype(jnp.float32), wo_my,
                                 preferred_element_type=jnp.float32)

        for k in (0, 2, 1):
            origin = lax.rem(my_pos + k + 1, N_DEV)
            wo_k = wo_ref[pl.ds(origin * HD, HD), :]
            for b in range(B):
                for p in range(NP):
                    make_desc(k, b, p, k + 1).wait_recv()
                out_ref[b] = out_ref[b] + jnp.dot(
                    ctx_recv[k, b].astype(jnp.float32), wo_k,
                    preferred_element_type=jnp.float32)

        for k in range(N_DEV - 1):
            for b in range(B):
                for p in range(NP):
                    make_desc(k, b, p, 3 - k).wait_send()

    return pl.pallas_call(
        body,
        out_shape=jax.ShapeDtypeStruct((B, SQ, D), jnp.float32),
        in_specs=[pl.BlockSpec(memory_space=pltpu.VMEM)] * 5,
        out_specs=pl.BlockSpec(memory_space=pltpu.VMEM),
        scratch_shapes=[
            pltpu.VMEM((B, SQ, HD), jnp.bfloat16),
            pltpu.VMEM((N_DEV - 1, B, SQ, HD), jnp.bfloat16),
            pltpu.SemaphoreType.DMA((N_DEV - 1, B, NP)),
            pltpu.SemaphoreType.DMA((N_DEV - 1, B, NP)),
        ],
        compiler_params=pltpu.CompilerParams(collective_id=0),
    )(xf, Wq_loc, Kh, Vh, Wo)


# device time: 14352 ns/iter; 1.3342x vs baseline; 1.0326x over previous
import jax
import jax.numpy as jnp
from jax import lax
from jax.experimental import pallas as pl
from jax.experimental.pallas import tpu as pltpu

N_DEV = 4
B, SQ, SKV, D = 2, 256, 256, 512
HL, DH = 4, 64
HD = HL * DH
BLK = 64
NP = HL // 2
PW = 2 * DH


def kernel(x, Wq, K_ext, V_ext, Wo):
    my = lax.axis_index("i")
    Wq_loc = lax.dynamic_slice_in_dim(Wq, my * HD, HD, axis=1)
    xf = x.reshape(B * SQ, D)
    Kh = K_ext.transpose(0, 2, 1, 3)
    Vh = V_ext.transpose(0, 2, 1, 3)

    def body(x_ref, wq_ref, k_ref, v_ref, wo_ref, out_ref,
             ctx_mine, ctx_recv, ssems, rsems):
        my_pos = lax.axis_index("i")

        barrier_sem = pltpu.get_barrier_semaphore()
        for d in (1, 2, 3):
            pl.semaphore_signal(
                barrier_sem, inc=1,
                device_id=(lax.rem(my_pos + d, N_DEV),),
                device_id_type=pl.DeviceIdType.MESH,
            )


        def make_desc(k, b, p, dev_offset):
            return pltpu.make_async_remote_copy(
                src_ref=ctx_mine.at[b, :, pl.ds(p * PW, PW)],
                dst_ref=ctx_recv.at[k, b, :, pl.ds(p * PW, PW)],
                send_sem=ssems.at[k, b, p],
                recv_sem=rsems.at[k, b, p],
                device_id=(lax.rem(my_pos + dev_offset, N_DEV),),
                device_id_type=pl.DeviceIdType.MESH,
            )

        def softmax_ctx(q, kk, vv):
            s = lax.dot_general(q, kk, (((1,), (1,)), ((), ())),
                                preferred_element_type=jnp.float32)
            w = jnp.exp(s * 0.125)
            w = (w / jnp.sum(w, axis=1, keepdims=True)).astype(jnp.bfloat16)
            return jnp.dot(w, vv, preferred_element_type=jnp.float32)

        first = True
        for b in range(B):
            q_b = jnp.dot(x_ref[pl.ds(b * SQ, SQ), :], wq_ref[...],
                          preferred_element_type=jnp.float32)
            for p in range(NP):
                pair_parts = []
                for h in (2 * p, 2 * p + 1):
                    qh = q_b[:, h * DH:(h + 1) * DH].astype(jnp.bfloat16)
                    kh = k_ref[b, h].astype(jnp.bfloat16)
                    vh = v_ref[b, h].astype(jnp.bfloat16)
                    qa = jnp.concatenate([qh[:BLK], qh[3 * BLK:]], axis=0)
                    ka = jnp.concatenate([kh[:BLK], kh[3 * BLK:]], axis=0)
                    va = jnp.concatenate([vh[:BLK], vh[3 * BLK:]], axis=0)
                    ctx_a = softmax_ctx(qa, ka, va)
                    ctx_b = softmax_ctx(qh[BLK:3 * BLK],
                                        kh[:3 * BLK], vh[:3 * BLK])
                    pair_parts.append(jnp.concatenate(
                        [ctx_a[:BLK], ctx_b, ctx_a[BLK:]], axis=0))
                ctx_mine[b, :, pl.ds(p * PW, PW)] = jnp.concatenate(
                    pair_parts, axis=1).astype(jnp.bfloat16)
                if first:
                    pl.semaphore_wait(barrier_sem, 3)
                    first = False
                for d in (1, 2, 3):
                    make_desc(3 - d, b, p, d).start()

        wo_my = wo_ref[pl.ds(my_pos * HD, HD), :].astype(jnp.bfloat16)
        for b in range(B):
            out_ref[b] = jnp.dot(ctx_mine[b], wo_my,
                                 preferred_element_type=jnp.float32)

        for k in (0, 2, 1):
            origin = lax.rem(my_pos + k + 1, N_DEV)
            wo_k = wo_ref[pl.ds(origin * HD, HD), :].astype(jnp.bfloat16)
            for b in range(B):
                for p in range(NP):
                    make_desc(k, b, p, k + 1).wait_recv()
                out_ref[b] = out_ref[b] + jnp.dot(
                    ctx_recv[k, b], wo_k,
                    preferred_element_type=jnp.float32)

        for k in range(N_DEV - 1):
            for b in range(B):
                for p in range(NP):
                    make_desc(k, b, p, 3 - k).wait_send()

    return pl.pallas_call(
        body,
        out_shape=jax.ShapeDtypeStruct((B, SQ, D), jnp.float32),
        in_specs=[pl.BlockSpec(memory_space=pltpu.VMEM)] * 5,
        out_specs=pl.BlockSpec(memory_space=pltpu.VMEM),
        scratch_shapes=[
            pltpu.VMEM((B, SQ, HD), jnp.bfloat16),
            pltpu.VMEM((N_DEV - 1, B, SQ, HD), jnp.bfloat16),
            pltpu.SemaphoreType.DMA((N_DEV - 1, B, NP)),
            pltpu.SemaphoreType.DMA((N_DEV - 1, B, NP)),
        ],
        compiler_params=pltpu.CompilerParams(collective_id=0),
    )(xf, Wq_loc, Kh, Vh, Wo)
